# Initial kernel scaffold; baseline (speedup 1.0000x reference)
import jax
import jax.numpy as jnp
from jax.experimental import pallas as pl


def kernel(x, mask, k_param, W_lin):
    raise NotImplementedError("write your pallas kernel here")


# speedup vs baseline: 2.0212x; 2.0212x over previous
import functools

import jax
import jax.numpy as jnp
from jax.experimental import pallas as pl

H = 3
K = 87
TAU = 1.0
D = 768


def _fused_kernel(x_ref, mask_ref, k_ref, f_ref, s_ref, *, bb, n):
    rows = bb * n
    x2 = x_ref[...]
    xn2 = jnp.sum(x2 * x2, axis=1, keepdims=True)

    prec = jax.lax.Precision.DEFAULT
    invs = []
    for h in range(H):
        kh = k_ref[h]
        kn2 = jnp.sum(kh * kh, axis=1)[None, :]
        g = jax.lax.dot_general(
            x2, kh, (((1,), (1,)), ((), ())),
            precision=prec, preferred_element_type=jnp.float32)
        d2 = jnp.maximum(xn2 + kn2 - 2.0 * g, 0.0)
        m = jnp.mean(d2, axis=1, keepdims=True)
        invs.append(1.0 / ((d2 / m) / TAU))

    s_pre = jnp.maximum(jnp.maximum(invs[0], invs[1]), invs[2])
    s_pre = s_pre * mask_ref[...] * 5.0
    z = s_pre - jnp.max(s_pre, axis=1, keepdims=True)
    e = jnp.exp(z)
    s = e / jnp.sum(e, axis=1, keepdims=True)
    s_ref[...] = s

    grp = jax.lax.broadcasted_iota(jnp.int32, (rows, 1), 0) // n
    neg_inf = jnp.float32(-jnp.inf)
    sel_pick = [jnp.zeros((rows, K), dtype=jnp.float32) for _ in range(H)]
    for b in range(bb):
        in_b = grp == b
        a = [jnp.max(jnp.where(in_b, invs[h], neg_inf), axis=0, keepdims=True)
             for h in range(H)]
        pick2 = a[2] > jnp.maximum(a[0], a[1])
        pick1 = jnp.logical_and(a[1] > a[0], jnp.logical_not(pick2))
        pick0 = jnp.logical_not(jnp.logical_or(pick1, pick2))
        fb = in_b.astype(jnp.float32)
        for h, p in enumerate((pick0, pick1, pick2)):
            sel_pick[h] = sel_pick[h] + fb * p.astype(jnp.float32)

    acc = jnp.zeros((rows, D), dtype=jnp.float32)
    for h in range(H):
        th = s * sel_pick[h]
        acc = acc + jax.lax.dot_general(
            th, k_ref[h], (((1,), (0,)), ((), ())),
            precision=prec, preferred_element_type=jnp.float32)
    f_ref[...] = acc


def kernel(x, mask, k_param, W_lin):
    B, N, _ = x.shape
    bb = 8
    rows = bb * N
    grid = (B // bb,)
    x2 = x.reshape(B * N, D)
    mask_f = mask.astype(jnp.float32).reshape(B * N, 1)

    f2, s2 = pl.pallas_call(
        functools.partial(_fused_kernel, bb=bb, n=N),
        grid=grid,
        in_specs=[
            pl.BlockSpec((rows, D), lambda i: (i, 0)),
            pl.BlockSpec((rows, 1), lambda i: (i, 0)),
            pl.BlockSpec((H, K, D), lambda i: (0, 0, 0)),
        ],
        out_specs=[
            pl.BlockSpec((rows, D), lambda i: (i, 0)),
            pl.BlockSpec((rows, K), lambda i: (i, 0)),
        ],
        out_shape=[
            jax.ShapeDtypeStruct((B * N, D), jnp.float32),
            jax.ShapeDtypeStruct((B * N, K), jnp.float32),
        ],
    )(x2, mask_f, k_param)

    k_out = jnp.transpose(k_param, (1, 0, 2)).reshape(-1, D)
    return f2.reshape(B, N, D), s2.reshape(B, N, K), k_out


# speedup vs baseline: 2.2080x; 1.0924x over previous
import functools

import jax
import jax.numpy as jnp
from jax.experimental import pallas as pl
from jax.experimental.pallas import tpu as pltpu

H = 3
K = 87
TAU = 1.0
D = 768


def _fused_kernel(x_ref, mask_ref, k_ref, f_ref, s_ref, *, bb, n):
    rows = bb * n
    x2 = x_ref[...]
    xn2 = jnp.sum(x2 * x2, axis=1, keepdims=True)

    prec = jax.lax.Precision.DEFAULT
    invs = []
    for h in range(H):
        kh = k_ref[h]
        kn2 = jnp.sum(kh * kh, axis=1)[None, :]
        g = jax.lax.dot_general(
            x2, kh, (((1,), (1,)), ((), ())),
            precision=prec, preferred_element_type=jnp.float32)
        d2 = jnp.maximum(xn2 + kn2 - 2.0 * g, 0.0)
        m = jnp.mean(d2, axis=1, keepdims=True)
        invs.append(1.0 / ((d2 / m) / TAU))

    s_pre = jnp.maximum(jnp.maximum(invs[0], invs[1]), invs[2])
    s_pre = s_pre * mask_ref[...] * 5.0
    z = s_pre - jnp.max(s_pre, axis=1, keepdims=True)
    e = jnp.exp(z)
    s = e / jnp.sum(e, axis=1, keepdims=True)
    s_ref[...] = s

    a = []
    for h in range(H):
        a.append(jnp.concatenate(
            [jnp.max(invs[h][b * n:(b + 1) * n, :], axis=0, keepdims=True)
             for b in range(bb)], axis=0))
    pick2 = a[2] > jnp.maximum(a[0], a[1])
    pick1 = jnp.logical_and(a[1] > a[0], jnp.logical_not(pick2))
    pick0 = jnp.logical_not(jnp.logical_or(pick1, pick2))
    picks = (pick0, pick1, pick2)

    grp = jax.lax.broadcasted_iota(jnp.int32, (rows, bb), 0) // n
    lane = jax.lax.broadcasted_iota(jnp.int32, (rows, bb), 1)
    expand = (grp == lane).astype(jnp.float32)

    acc = jnp.zeros((rows, D), dtype=jnp.float32)
    for h in range(H):
        sel = jax.lax.dot_general(
            expand, picks[h].astype(jnp.float32), (((1,), (0,)), ((), ())),
            precision=prec, preferred_element_type=jnp.float32)
        acc = acc + jax.lax.dot_general(
            s * sel, k_ref[h], (((1,), (0,)), ((), ())),
            precision=prec, preferred_element_type=jnp.float32)
    f_ref[...] = acc


def kernel(x, mask, k_param, W_lin):
    B, N, _ = x.shape
    bb = 8
    rows = bb * N
    grid = (B // bb,)
    x2 = x.reshape(B * N, D)
    mask_f = mask.astype(jnp.float32).reshape(B * N, 1)

    f2, s2 = pl.pallas_call(
        functools.partial(_fused_kernel, bb=bb, n=N),
        grid=grid,
        in_specs=[
            pl.BlockSpec((rows, D), lambda i: (i, 0)),
            pl.BlockSpec((rows, 1), lambda i: (i, 0)),
            pl.BlockSpec((H, K, D), lambda i: (0, 0, 0)),
        ],
        out_specs=[
            pl.BlockSpec((rows, D), lambda i: (i, 0)),
            pl.BlockSpec((rows, K), lambda i: (i, 0)),
        ],
        out_shape=[
            jax.ShapeDtypeStruct((B * N, D), jnp.float32),
            jax.ShapeDtypeStruct((B * N, K), jnp.float32),
        ],
        compiler_params=pltpu.CompilerParams(
            dimension_semantics=("parallel",),
        ),
    )(x2, mask_f, k_param)

    k_out = jnp.transpose(k_param, (1, 0, 2)).reshape(-1, D)
    return f2.reshape(B, N, D), s2.reshape(B, N, K), k_out


# speedup vs baseline: 2.3055x; 1.0442x over previous
import functools

import jax
import jax.numpy as jnp
from jax.experimental import pallas as pl
from jax.experimental.pallas import tpu as pltpu

H = 3
K = 87
TAU = 1.0
D = 768


def _fused_kernel(x_ref, mask_ref, k_ref, f_ref, s_ref, *, bb, n):
    rows = bb * n
    x2 = x_ref[...]
    xn2 = jnp.sum(x2 * x2, axis=1, keepdims=True)

    prec = jax.lax.Precision.DEFAULT
    invs = []
    for h in range(H):
        kh = k_ref[h]
        kn2 = jnp.sum(kh * kh, axis=1)[None, :]
        g = jax.lax.dot_general(
            x2, kh, (((1,), (1,)), ((), ())),
            precision=prec, preferred_element_type=jnp.float32)
        d2 = jnp.maximum(xn2 + kn2 - 2.0 * g, 0.0)
        m = jnp.mean(d2, axis=1, keepdims=True)
        invs.append(1.0 / ((d2 / m) / TAU))

    s_pre = jnp.maximum(jnp.maximum(invs[0], invs[1]), invs[2])
    s_pre = s_pre * mask_ref[...] * 5.0
    z = s_pre - jnp.max(s_pre, axis=1, keepdims=True)
    e = jnp.exp(z)
    s = e / jnp.sum(e, axis=1, keepdims=True)
    s_ref[...] = s

    a = []
    for h in range(H):
        a.append(jnp.concatenate(
            [jnp.max(invs[h][b * n:(b + 1) * n, :], axis=0, keepdims=True)
             for b in range(bb)], axis=0))
    pick2 = a[2] > jnp.maximum(a[0], a[1])
    pick1 = jnp.logical_and(a[1] > a[0], jnp.logical_not(pick2))
    pick0 = jnp.logical_not(jnp.logical_or(pick1, pick2))
    picks = (pick0, pick1, pick2)

    grp = jax.lax.broadcasted_iota(jnp.int32, (rows, bb), 0) // n
    lane = jax.lax.broadcasted_iota(jnp.int32, (rows, bb), 1)
    expand = (grp == lane).astype(jnp.float32)

    acc = jnp.zeros((rows, D), dtype=jnp.float32)
    for h in range(H):
        sel = jax.lax.dot_general(
            expand, picks[h].astype(jnp.float32), (((1,), (0,)), ((), ())),
            precision=prec, preferred_element_type=jnp.float32)
        acc = acc + jax.lax.dot_general(
            s * sel, k_ref[h], (((1,), (0,)), ((), ())),
            precision=prec, preferred_element_type=jnp.float32)
    f_ref[...] = acc


def kernel(x, mask, k_param, W_lin):
    B, N, _ = x.shape
    bb = 16
    rows = bb * N
    grid = (B // bb,)
    x2 = x.reshape(B * N, D)
    mask_f = mask.astype(jnp.float32).reshape(B * N, 1)

    f2, s2 = pl.pallas_call(
        functools.partial(_fused_kernel, bb=bb, n=N),
        grid=grid,
        in_specs=[
            pl.BlockSpec((rows, D), lambda i: (i, 0)),
            pl.BlockSpec((rows, 1), lambda i: (i, 0)),
            pl.BlockSpec((H, K, D), lambda i: (0, 0, 0)),
        ],
        out_specs=[
            pl.BlockSpec((rows, D), lambda i: (i, 0)),
            pl.BlockSpec((rows, K), lambda i: (i, 0)),
        ],
        out_shape=[
            jax.ShapeDtypeStruct((B * N, D), jnp.float32),
            jax.ShapeDtypeStruct((B * N, K), jnp.float32),
        ],
        compiler_params=pltpu.CompilerParams(
            dimension_semantics=("parallel",),
        ),
    )(x2, mask_f, k_param)

    k_out = jnp.transpose(k_param, (1, 0, 2)).reshape(-1, D)
    return f2.reshape(B, N, D), s2.reshape(B, N, K), k_out


# speedup vs baseline: 2.3451x; 1.0171x over previous
import functools

import jax
import jax.numpy as jnp
from jax.experimental import pallas as pl
from jax.experimental.pallas import tpu as pltpu

H = 3
K = 87
TAU = 1.0
D = 768


def _fused_kernel(x_ref, mask_ref, k_ref, f_ref, s_ref, *, bb, n):
    rows = bb * n
    x2 = x_ref[...]
    xn2 = jnp.sum(x2 * x2, axis=1, keepdims=True)

    prec = jax.lax.Precision.DEFAULT
    invs = []
    for h in range(H):
        kh = k_ref[h]
        kn2 = jnp.sum(kh * kh, axis=1)[None, :]
        g = jax.lax.dot_general(
            x2, kh, (((1,), (1,)), ((), ())),
            precision=prec, preferred_element_type=jnp.float32)
        d2 = jnp.maximum(xn2 + kn2 - 2.0 * g, 0.0)
        m = jnp.mean(d2, axis=1, keepdims=True)
        invs.append(1.0 / ((d2 / m) / TAU))

    s_pre = jnp.maximum(jnp.maximum(invs[0], invs[1]), invs[2])
    s_pre = s_pre * mask_ref[...] * 5.0
    z = s_pre - jnp.max(s_pre, axis=1, keepdims=True)
    e = jnp.exp(z)
    s = e / jnp.sum(e, axis=1, keepdims=True)
    s_ref[...] = s

    a = []
    for h in range(H):
        a.append(jnp.concatenate(
            [jnp.max(invs[h][b * n:(b + 1) * n, :], axis=0, keepdims=True)
             for b in range(bb)], axis=0))
    pick2 = a[2] > jnp.maximum(a[0], a[1])
    pick1 = jnp.logical_and(a[1] > a[0], jnp.logical_not(pick2))
    pick0 = jnp.logical_not(jnp.logical_or(pick1, pick2))
    picks = (pick0, pick1, pick2)

    grp = jax.lax.broadcasted_iota(jnp.int32, (rows, bb), 0) // n
    lane = jax.lax.broadcasted_iota(jnp.int32, (rows, bb), 1)
    expand = (grp == lane).astype(jnp.float32)

    acc = jnp.zeros((rows, D), dtype=jnp.float32)
    for h in range(H):
        sel = jax.lax.dot_general(
            expand, picks[h].astype(jnp.float32), (((1,), (0,)), ((), ())),
            precision=prec, preferred_element_type=jnp.float32)
        acc = acc + jax.lax.dot_general(
            s * sel, k_ref[h], (((1,), (0,)), ((), ())),
            precision=prec, preferred_element_type=jnp.float32)
    f_ref[...] = acc


def kernel(x, mask, k_param, W_lin):
    B, N, _ = x.shape
    bb = 32
    rows = bb * N
    grid = (B // bb,)
    x2 = x.reshape(B * N, D)
    mask_f = mask.astype(jnp.float32).reshape(B * N, 1)

    f2, s2 = pl.pallas_call(
        functools.partial(_fused_kernel, bb=bb, n=N),
        grid=grid,
        in_specs=[
            pl.BlockSpec((rows, D), lambda i: (i, 0)),
            pl.BlockSpec((rows, 1), lambda i: (i, 0)),
            pl.BlockSpec((H, K, D), lambda i: (0, 0, 0)),
        ],
        out_specs=[
            pl.BlockSpec((rows, D), lambda i: (i, 0)),
            pl.BlockSpec((rows, K), lambda i: (i, 0)),
        ],
        out_shape=[
            jax.ShapeDtypeStruct((B * N, D), jnp.float32),
            jax.ShapeDtypeStruct((B * N, K), jnp.float32),
        ],
        compiler_params=pltpu.CompilerParams(
            dimension_semantics=("parallel",),
        ),
    )(x2, mask_f, k_param)

    k_out = jnp.transpose(k_param, (1, 0, 2)).reshape(-1, D)
    return f2.reshape(B, N, D), s2.reshape(B, N, K), k_out
